# Initial kernel scaffold; baseline (speedup 1.0000x reference)
import functools

import jax
import jax.numpy as jnp
from jax import lax
from jax.experimental import pallas as pl
from jax.experimental.pallas import tpu as pltpu, tpu_sc as plsc

NC = 2
NS = 16
NW = NC * NS

CHUNK = 128
GATHER = CHUNK * 8


def _make_sc_kernel(n_rows, chars, vocab, dim):
    rows_per_w = n_rows // NW
    n_chunks = rows_per_w // CHUNK
    idx_rows_per_chunk = (CHUNK * chars) // 128

    mesh = plsc.VectorSubcoreMesh(core_axis_name="c", subcore_axis_name="s")

    @functools.partial(
        pl.kernel,
        mesh=mesh,
        out_type=jax.ShapeDtypeStruct((n_rows, dim), jnp.float32),
        scratch_types=[
            pltpu.VMEM((idx_rows_per_chunk, 128), jnp.int32),
            pltpu.VMEM((GATHER, dim), jnp.float32),
            pltpu.VMEM((CHUNK, dim), jnp.float32),
            pltpu.SemaphoreType.DMA,
        ],
    )
    def embed_sum(idx_hbm, table_hbm, out_hbm, idx_v, rows_v, out_v, sem):
        wid = lax.axis_index("s") * NC + lax.axis_index("c")

        def chunk_body(g, carry):
            base = wid * rows_per_w + g * CHUNK
            irow0 = (base * chars) // 128
            pltpu.sync_copy(idx_hbm.at[pl.ds(irow0, idx_rows_per_chunk)], idx_v)
            copies = []
            for r in range(idx_rows_per_chunk):
                cp = pltpu.async_copy(
                    table_hbm.at[idx_v.at[r]],
                    rows_v.at[pl.ds(r * 128, 128)],
                    sem,
                )
                copies.append(cp)
            for cp in copies:
                cp.wait()

            def row_body(c, carry2):
                for d in range(dim // 16):
                    sl = pl.ds(d * 16, 16)
                    acc = rows_v[c * chars, sl]
                    for j in range(1, chars):
                        acc = acc + rows_v[c * chars + j, sl]
                out_v[c, sl] = acc
                return carry2

            lax.fori_loop(0, CHUNK, row_body, 0, unroll=2)
            pltpu.sync_copy(out_v, out_hbm.at[pl.ds(base, CHUNK)])
            return carry

        lax.fori_loop(0, n_chunks, chunk_body, 0)

    return embed_sum


def kernel(morphemes, table):
    b, s, chars = morphemes.shape
    vocab, dim = table.shape
    n_rows = b * s
    idx2d = morphemes.reshape((n_rows * chars) // 128, 128)
    fn = _make_sc_kernel(n_rows, chars, vocab, dim)
    out = fn(idx2d, table)
    return out.reshape(b, s, dim)


# speedup vs baseline: 8.4200x; 8.4200x over previous
import functools

import jax
import jax.numpy as jnp
from jax import lax
from jax.experimental import pallas as pl
from jax.experimental.pallas import tpu as pltpu, tpu_sc as plsc

NC = 2
NS = 16
NW = NC * NS

CHUNK = 128
GATHER = CHUNK * 8


def _make_sc_kernel(n_rows, chars, vocab, dim):
    rows_per_w = n_rows // NW
    n_chunks = rows_per_w // CHUNK
    idx_rows_per_chunk = (CHUNK * chars) // 128

    mesh = plsc.VectorSubcoreMesh(core_axis_name="c", subcore_axis_name="s")

    @functools.partial(
        pl.kernel,
        mesh=mesh,
        compiler_params=pltpu.CompilerParams(use_tc_tiling_on_sc=False),
        out_type=jax.ShapeDtypeStruct((n_rows, dim), jnp.float32),
        scratch_types=[
            pltpu.VMEM((idx_rows_per_chunk, 128), jnp.int32),
            pltpu.VMEM((GATHER, dim), jnp.float32),
            pltpu.VMEM((CHUNK, dim), jnp.float32),
            pltpu.SemaphoreType.DMA,
        ],
    )
    def embed_sum(idx_hbm, table_hbm, out_hbm, idx_v, rows_v, out_v, sem):
        wid = lax.axis_index("s") * NC + lax.axis_index("c")

        def chunk_body(g, carry):
            base = pl.multiple_of(wid * rows_per_w + g * CHUNK, CHUNK)
            irow0 = pl.multiple_of((base * chars) // 128, idx_rows_per_chunk)
            pltpu.sync_copy(idx_hbm.at[pl.ds(irow0, idx_rows_per_chunk)], idx_v)
            copies = []
            for r in range(idx_rows_per_chunk):
                cp = pltpu.async_copy(
                    table_hbm.at[idx_v.at[r]],
                    rows_v.at[pl.ds(r * 128, 128)],
                    sem,
                )
                copies.append(cp)
            for cp in copies:
                cp.wait()

            def row_body(c, carry2):
                for d in range(dim // 16):
                    sl = pl.ds(d * 16, 16)
                    acc = rows_v[c * chars, sl]
                    for j in range(1, chars):
                        acc = acc + rows_v[c * chars + j, sl]
                    out_v[c, sl] = acc
                return carry2

            lax.fori_loop(0, CHUNK, row_body, 0, unroll=2)
            pltpu.sync_copy(out_v, out_hbm.at[pl.ds(base, CHUNK)])
            return carry

        lax.fori_loop(0, n_chunks, chunk_body, 0)

    return embed_sum


def kernel(morphemes, table):
    b, s, chars = morphemes.shape
    vocab, dim = table.shape
    n_rows = b * s
    idx2d = morphemes.reshape((n_rows * chars) // 128, 128)
    fn = _make_sc_kernel(n_rows, chars, vocab, dim)
    out = fn(idx2d, table)
    return out.reshape(b, s, dim)


# speedup vs baseline: 10.3065x; 1.2241x over previous
import functools

import jax
import jax.numpy as jnp
from jax import lax
from jax.experimental import pallas as pl
from jax.experimental.pallas import tpu as pltpu, tpu_sc as plsc

NC = 2
NS = 16
NW = NC * NS

CHUNK = 64
GATHER = CHUNK * 8


def _make_sc_kernel(n_rows, chars, vocab, dim):
    rows_per_w = n_rows // NW
    n_chunks = rows_per_w // CHUNK
    assert n_chunks % 2 == 0
    idx_rows = (CHUNK * chars) // 128

    mesh = plsc.VectorSubcoreMesh(core_axis_name="c", subcore_axis_name="s")

    @functools.partial(
        pl.kernel,
        mesh=mesh,
        compiler_params=pltpu.CompilerParams(use_tc_tiling_on_sc=False),
        out_type=jax.ShapeDtypeStruct((n_rows, dim), jnp.float32),
        scratch_types=[
            pltpu.VMEM((2, idx_rows, 128), jnp.int32),
            pltpu.VMEM((2, GATHER, dim), jnp.float32),
            pltpu.VMEM((2, CHUNK, dim), jnp.float32),
            pltpu.SemaphoreType.DMA,
            pltpu.SemaphoreType.DMA,
            pltpu.SemaphoreType.DMA,
            pltpu.SemaphoreType.DMA,
        ],
    )
    def embed_sum(idx_hbm, table_hbm, out_hbm, idx_v, rows_v, out_v,
                  sem_ga, sem_gb, sem_oa, sem_ob):
        wid = lax.axis_index("s") * NC + lax.axis_index("c")
        sem_g = [sem_ga, sem_gb]
        sem_o = [sem_oa, sem_ob]

        def base_of(g):
            return pl.multiple_of(wid * rows_per_w + g * CHUNK, CHUNK)

        def gather_copies(g, b):
            irow0 = pl.multiple_of((base_of(g) * chars) // 128, idx_rows)
            idx_cp = pltpu.make_async_copy(
                idx_hbm.at[pl.ds(irow0, idx_rows)], idx_v.at[b], sem_g[b])
            row_cps = [
                pltpu.make_async_copy(
                    table_hbm.at[idx_v.at[b].at[r]],
                    rows_v.at[b].at[pl.ds(r * 128, 128)],
                    sem_g[b])
                for r in range(idx_rows)
            ]
            return idx_cp, row_cps

        def start_load(g, b):
            idx_cp, row_cps = gather_copies(g, b)
            idx_cp.start()
            idx_cp.wait()
            for cp in row_cps:
                cp.start()

        def wait_load(g, b):
            _, row_cps = gather_copies(g, b)
            for cp in row_cps:
                cp.wait()

        def out_copy(g, b):
            return pltpu.make_async_copy(
                out_v.at[b], out_hbm.at[pl.ds(base_of(g), CHUNK)], sem_o[b])

        start_load(0, 0)

        def pair_body(gg, carry):
            for b in range(2):
                g = gg * 2 + b
                nb = 1 - b

                @pl.when(g + 1 < n_chunks)
                def _():
                    start_load(g + 1, nb)

                wait_load(g, b)

                @pl.when(g >= 2)
                def _():
                    out_copy(g - 2, b).wait()

                rv = rows_v.at[b]
                ov = out_v.at[b]

                def row_body(c, carry2):
                    for d in range(dim // 16):
                        sl = pl.ds(d * 16, 16)
                        acc = rv[c * chars, sl]
                        for j in range(1, chars):
                            acc = acc + rv[c * chars + j, sl]
                        ov[c, sl] = acc
                    return carry2

                lax.fori_loop(0, CHUNK, row_body, 0, unroll=2)
                out_copy(g, b).start()
            return carry

        lax.fori_loop(0, n_chunks // 2, pair_body, 0)
        for b in range(2):
            out_copy(n_chunks - 2 + b, b).wait()

    return embed_sum


def kernel(morphemes, table):
    b, s, chars = morphemes.shape
    vocab, dim = table.shape
    n_rows = b * s
    idx2d = morphemes.reshape((n_rows * chars) // 128, 128)
    fn = _make_sc_kernel(n_rows, chars, vocab, dim)
    out = fn(idx2d, table)
    return out.reshape(b, s, dim)


# speedup vs baseline: 16.5556x; 1.6063x over previous
import functools

import jax
import jax.numpy as jnp
from jax import lax
from jax.experimental import pallas as pl
from jax.experimental.pallas import tpu as pltpu, tpu_sc as plsc

NC = 2
NS = 16
NW = NC * NS

CHUNK = 128


def _make_sc_kernel(n_rows, chars, vocab, dim):
    rows_per_w = n_rows // NW
    n_chunks = rows_per_w // CHUNK
    assert n_chunks % 2 == 0
    idx_rows = (CHUNK * chars) // 128
    w_idx_rows = n_chunks * idx_rows

    mesh = plsc.VectorSubcoreMesh(core_axis_name="c", subcore_axis_name="s")

    @functools.partial(
        pl.kernel,
        mesh=mesh,
        compiler_params=pltpu.CompilerParams(use_tc_tiling_on_sc=False),
        out_type=jax.ShapeDtypeStruct((n_rows, dim), jnp.float32),
        scratch_types=[
            pltpu.VMEM((w_idx_rows, 128), jnp.int32),
            pltpu.VMEM((2, CHUNK, dim), jnp.float32),
            pltpu.SemaphoreType.DMA,
            pltpu.SemaphoreType.DMA,
            pltpu.SemaphoreType.DMA,
            pltpu.SemaphoreType.DMA,
            pltpu.SemaphoreType.DMA,
        ],
    )
    def embed_sum(idx_hbm, table_hbm, out_hbm, idx_v, acc_v,
                  sem_i, sem_g0, sem_g1, sem_o0, sem_o1):
        wid = lax.axis_index("s") * NC + lax.axis_index("c")
        sem_g = [sem_g0, sem_g1]
        sem_o = [sem_o0, sem_o1]

        irow0 = pl.multiple_of(wid * w_idx_rows, 8)
        pltpu.sync_copy(idx_hbm.at[pl.ds(irow0, w_idx_rows)], idx_v)

        def base_of(g):
            return pl.multiple_of(wid * rows_per_w + g * CHUNK, CHUNK)

        def start_gathers(g, b):
            for j in range(idx_rows):
                pltpu.async_copy(
                    table_hbm.at[idx_v.at[g * idx_rows + j]],
                    acc_v.at[b],
                    sem_g[b],
                    add=True,
                )

        def wait_gathers(b):
            for _ in range(idx_rows):
                pltpu.make_async_copy(
                    table_hbm.at[idx_v.at[0]], acc_v.at[b], sem_g[b]).wait()

        def out_copy(g, b):
            return pltpu.make_async_copy(
                acc_v.at[b], out_hbm.at[pl.ds(base_of(g), CHUNK)], sem_o[b])

        zero = jnp.zeros((16,), jnp.float32)

        def zero_acc(b):
            av = acc_v.at[b]

            def zb(c, carry):
                for d in range(dim // 16):
                    av[c, pl.ds(d * 16, 16)] = zero
                return carry

            lax.fori_loop(0, CHUNK, zb, 0, unroll=4)

        def pair_body(gg, carry):
            for b in range(2):
                g = gg * 2 + b
                nb = 1 - b

                @pl.when(g >= 2)
                def _():
                    out_copy(g - 2, b).wait()

                zero_acc(b)
                start_gathers(g, b)

                @pl.when(g >= 1)
                def _():
                    wait_gathers(nb)
                    out_copy(g - 1, nb).start()

            return carry

        lax.fori_loop(0, n_chunks // 2, pair_body, 0)
        wait_gathers(1)
        out_copy(n_chunks - 1, 1).start()
        out_copy(n_chunks - 2, 0).wait()
        out_copy(n_chunks - 1, 1).wait()

    return embed_sum


def kernel(morphemes, table):
    b, s, chars = morphemes.shape
    vocab, dim = table.shape
    n_rows = b * s
    idx2d = (
        morphemes.reshape(n_rows // 128, 128, chars)
        .transpose(0, 2, 1)
        .reshape((n_rows * chars) // 128, 128)
    )
    fn = _make_sc_kernel(n_rows, chars, vocab, dim)
    out = fn(idx2d, table)
    return out.reshape(b, s, dim)
